# Initial kernel scaffold; baseline (speedup 1.0000x reference)
import jax
import jax.numpy as jnp
from jax.experimental import pallas as pl


def kernel(energies, scale, shift, global_scale, global_shift, species, graph_i, n_node):
    raise NotImplementedError("write your pallas kernel here")


# speedup vs baseline: 22.8705x; 22.8705x over previous
import functools
import math

import jax
import jax.numpy as jnp
from jax import lax
from jax.experimental import pallas as pl
from jax.experimental.pallas import tpu as pltpu
from jax.experimental.pallas import tpu_sc as plsc

_N_NODES = 50000
_N_GRAPHS = 512
_NW = 32
_CHUNK = 1568
_PAD_N = _NW * _CHUNK
_ROW = 528
_LANES = 16

def _sc_partials_body(e_hbm, sp_hbm, g_hbm, scale_hbm, shift_hbm, out_hbm,
                      e_v, sp_v, g_v, scale_v, shift_v,
                      acc_sc, acc_sh, acc_e, res_v):
    wid = lax.axis_index("s") * 2 + lax.axis_index("c")
    base = wid * _CHUNK
    pltpu.sync_copy(e_hbm.at[pl.ds(base, _CHUNK)], e_v)
    pltpu.sync_copy(sp_hbm.at[pl.ds(base, _CHUNK)], sp_v)
    pltpu.sync_copy(g_hbm.at[pl.ds(base, _CHUNK)], g_v)
    pltpu.sync_copy(scale_hbm, scale_v)
    pltpu.sync_copy(shift_hbm, shift_v)

    zero = jnp.zeros((_LANES,), jnp.float32)

    def zbody(j, carry):
        sl = pl.ds(j * _LANES, _LANES)
        acc_sc[sl] = zero
        acc_sh[sl] = zero
        acc_e[sl] = zero
        return carry

    lax.fori_loop(0, _ROW, zbody, 0)

    lane_off = lax.iota(jnp.int32, _LANES) * _ROW

    def body(i, carry):
        sl = pl.ds(i * _LANES, _LANES)
        addr = lane_off + g_v[sl]
        plsc.addupdate_scatter(acc_sc, [addr], plsc.load_gather(scale_v, [sp_v[sl]]))
        plsc.addupdate_scatter(acc_sh, [addr], plsc.load_gather(shift_v, [sp_v[sl]]))
        plsc.addupdate_scatter(acc_e, [addr], e_v[sl])
        return carry

    lax.fori_loop(0, _CHUNK // _LANES, body, 0)

    for q, acc in enumerate((acc_sc, acc_sh, acc_e)):
        def rbody(j, carry, acc=acc):
            s = acc[pl.ds(j * _LANES, _LANES)]
            for lane in range(1, _LANES):
                s = s + acc[pl.ds(lane * _ROW + j * _LANES, _LANES)]
            res_v[pl.ds(j * _LANES, _LANES)] = s
            return carry

        lax.fori_loop(0, _N_GRAPHS // _LANES, rbody, 0)
        pltpu.sync_copy(res_v, out_hbm.at[q * _NW + wid])


@functools.cache
def _build_sc_partials():
    mesh = plsc.VectorSubcoreMesh(core_axis_name="c", subcore_axis_name="s")
    return pl.kernel(
        _sc_partials_body,
        out_type=jax.ShapeDtypeStruct((3 * _NW, _N_GRAPHS), jnp.float32),
        mesh=mesh,
        compiler_params=pltpu.CompilerParams(needs_layout_passes=False),
        scratch_types=[
            pltpu.VMEM((_CHUNK,), jnp.float32),
            pltpu.VMEM((_CHUNK,), jnp.int32),
            pltpu.VMEM((_CHUNK,), jnp.int32),
            pltpu.VMEM((128,), jnp.float32),
            pltpu.VMEM((128,), jnp.float32),
            pltpu.VMEM((_LANES * _ROW,), jnp.float32),
            pltpu.VMEM((_LANES * _ROW,), jnp.float32),
            pltpu.VMEM((_LANES * _ROW,), jnp.float32),
            pltpu.VMEM((_N_GRAPHS,), jnp.float32),
        ],
    )


def _tc_finish(part_ref, n_ref, gs_ref, gsh_ref, out_ref):
    p = part_ref[...]
    sc_sum = jnp.sum(p[0:_NW, :], axis=0, keepdims=True)
    sh_sum = jnp.sum(p[_NW:2 * _NW, :], axis=0, keepdims=True)
    e_sum = jnp.sum(p[2 * _NW:3 * _NW, :], axis=0, keepdims=True)
    num_atoms = jnp.maximum(n_ref[...].astype(jnp.float32), 1.0)
    c = math.log(math.e - 1.0)
    sc = jax.nn.softplus(sc_sum + c) / num_atoms * jax.nn.softplus(gs_ref[0] + c)
    sh = sh_sum / num_atoms + gsh_ref[0]
    out_ref[...] = (e_sum / num_atoms) * sc + sh


def kernel(energies, scale, shift, global_scale, global_shift, species, graph_i, n_node):
    pad = _PAD_N - _N_NODES
    e_pad = jnp.concatenate([energies, jnp.zeros((pad,), jnp.float32)])
    sp_pad = jnp.concatenate([species.astype(jnp.int32),
                              jnp.zeros((pad,), jnp.int32)])
    g_pad = jnp.concatenate([graph_i.astype(jnp.int32),
                             jnp.full((pad,), _N_GRAPHS, jnp.int32)])
    scale_pad = jnp.concatenate([scale, jnp.zeros((128 - scale.shape[0],), jnp.float32)])
    shift_pad = jnp.concatenate([shift, jnp.zeros((128 - shift.shape[0],), jnp.float32)])

    partials = _build_sc_partials()(e_pad, sp_pad, g_pad, scale_pad, shift_pad)

    out = pl.pallas_call(
        _tc_finish,
        out_shape=jax.ShapeDtypeStruct((1, _N_GRAPHS), jnp.float32),
        in_specs=[
            pl.BlockSpec(memory_space=pltpu.VMEM),
            pl.BlockSpec(memory_space=pltpu.VMEM),
            pl.BlockSpec(memory_space=pltpu.SMEM),
            pl.BlockSpec(memory_space=pltpu.SMEM),
        ],
        out_specs=pl.BlockSpec(memory_space=pltpu.VMEM),
    )(partials, n_node.reshape(1, _N_GRAPHS), global_scale, global_shift)

    return out.reshape(_N_GRAPHS, 1)


# speedup vs baseline: 30.0999x; 1.3161x over previous
import functools
import math

import jax
import jax.numpy as jnp
from jax import lax
from jax.experimental import pallas as pl
from jax.experimental.pallas import tpu as pltpu
from jax.experimental.pallas import tpu_sc as plsc

_N_NODES = 50000
_N_GRAPHS = 512
_N_SPECIES = 119
_NW = 32
_CHUNK = 1568
_TAIL = _N_NODES - 31 * _CHUNK
_EXTRA = _CHUNK - _TAIL
_ROW = 529
_LANES = 16
_ACC = 8512


def _sc_partials_body(e_hbm, sp_hbm, g_hbm, scale_hbm, shift_hbm, out_hbm,
                      e_v, sp_v, g_v, scale_v, shift_v,
                      acc_sc, acc_sh, acc_e, res_sc, res_sh, res_e,
                      sem_in, sem_out):
    wid = lax.axis_index("s") * 2 + lax.axis_index("c")
    base = wid * _CHUNK
    not_last = wid != _NW - 1

    pltpu.async_copy(e_hbm.at[pl.ds(base, _TAIL)], e_v.at[pl.ds(0, _TAIL)], sem_in)
    pltpu.async_copy(sp_hbm.at[pl.ds(base, _TAIL)], sp_v.at[pl.ds(0, _TAIL)], sem_in)
    pltpu.async_copy(g_hbm.at[pl.ds(base, _TAIL)], g_v.at[pl.ds(0, _TAIL)], sem_in)
    pltpu.async_copy(scale_hbm, scale_v, sem_in)
    pltpu.async_copy(shift_hbm, shift_v, sem_in)

    @pl.when(not_last)
    def _fire_extra():
        pltpu.async_copy(e_hbm.at[pl.ds(base + _TAIL, _EXTRA)],
                         e_v.at[pl.ds(_TAIL, _EXTRA)], sem_in)
        pltpu.async_copy(sp_hbm.at[pl.ds(base + _TAIL, _EXTRA)],
                         sp_v.at[pl.ds(_TAIL, _EXTRA)], sem_in)
        pltpu.async_copy(g_hbm.at[pl.ds(base + _TAIL, _EXTRA)],
                         g_v.at[pl.ds(_TAIL, _EXTRA)], sem_in)

    zero = jnp.zeros((_LANES,), jnp.float32)

    @plsc.parallel_loop(0, _ACC // _LANES, unroll=4)
    def _zero(j):
        sl = pl.ds(j * _LANES, _LANES)
        acc_sc[sl] = zero
        acc_sh[sl] = zero
        acc_e[sl] = zero

    pltpu.make_async_copy(e_hbm.at[pl.ds(base, _TAIL)], e_v.at[pl.ds(0, _TAIL)], sem_in).wait()
    pltpu.make_async_copy(sp_hbm.at[pl.ds(base, _TAIL)], sp_v.at[pl.ds(0, _TAIL)], sem_in).wait()
    pltpu.make_async_copy(g_hbm.at[pl.ds(base, _TAIL)], g_v.at[pl.ds(0, _TAIL)], sem_in).wait()
    pltpu.make_async_copy(scale_hbm, scale_v, sem_in).wait()
    pltpu.make_async_copy(shift_hbm, shift_v, sem_in).wait()

    @pl.when(not_last)
    def _drain_extra():
        pltpu.make_async_copy(e_hbm.at[pl.ds(base + _TAIL, _EXTRA)],
                              e_v.at[pl.ds(_TAIL, _EXTRA)], sem_in).wait()
        pltpu.make_async_copy(sp_hbm.at[pl.ds(base + _TAIL, _EXTRA)],
                              sp_v.at[pl.ds(_TAIL, _EXTRA)], sem_in).wait()
        pltpu.make_async_copy(g_hbm.at[pl.ds(base + _TAIL, _EXTRA)],
                              g_v.at[pl.ds(_TAIL, _EXTRA)], sem_in).wait()

    lane_off = lax.iota(jnp.int32, _LANES) * _ROW

    def step(i, carry):
        sl = pl.ds(i * _LANES, _LANES)
        addr = lane_off + g_v[sl]
        plsc.addupdate_scatter(acc_sc, [addr], plsc.load_gather(scale_v, [sp_v[sl]]))
        plsc.addupdate_scatter(acc_sh, [addr], plsc.load_gather(shift_v, [sp_v[sl]]))
        plsc.addupdate_scatter(acc_e, [addr], e_v[sl])
        return carry

    lax.fori_loop(0, _TAIL // _LANES, step, 0)

    @pl.when(not_last)
    def _steps_extra():
        lax.fori_loop(_TAIL // _LANES, _CHUNK // _LANES, step, 0)

    for q, (acc, res) in enumerate(
        ((acc_sc, res_sc), (acc_sh, res_sh), (acc_e, res_e))
    ):
        @plsc.parallel_loop(0, _N_GRAPHS // _LANES, unroll=2)
        def _reduce(j, acc=acc, res=res):
            off = j * _LANES
            vals = [acc[pl.ds(lane * _ROW + off, _LANES)] for lane in range(_LANES)]
            while len(vals) > 1:
                vals = [a + b for a, b in zip(vals[::2], vals[1::2])]
            res[pl.ds(off, _LANES)] = vals[0]

        pltpu.async_copy(res, out_hbm.at[q * _NW + wid], sem_out)

    pltpu.make_async_copy(res_sc, out_hbm.at[wid], sem_out).wait()
    pltpu.make_async_copy(res_sh, out_hbm.at[_NW + wid], sem_out).wait()
    pltpu.make_async_copy(res_e, out_hbm.at[2 * _NW + wid], sem_out).wait()


@functools.cache
def _build_sc_partials():
    mesh = plsc.VectorSubcoreMesh(core_axis_name="c", subcore_axis_name="s")
    return pl.kernel(
        _sc_partials_body,
        out_type=jax.ShapeDtypeStruct((3 * _NW, _N_GRAPHS), jnp.float32),
        mesh=mesh,
        compiler_params=pltpu.CompilerParams(needs_layout_passes=False),
        scratch_types=[
            pltpu.VMEM((_CHUNK,), jnp.float32),
            pltpu.VMEM((_CHUNK,), jnp.int32),
            pltpu.VMEM((_CHUNK,), jnp.int32),
            pltpu.VMEM((_N_SPECIES,), jnp.float32),
            pltpu.VMEM((_N_SPECIES,), jnp.float32),
            pltpu.VMEM((_ACC,), jnp.float32),
            pltpu.VMEM((_ACC,), jnp.float32),
            pltpu.VMEM((_ACC,), jnp.float32),
            pltpu.VMEM((_N_GRAPHS,), jnp.float32),
            pltpu.VMEM((_N_GRAPHS,), jnp.float32),
            pltpu.VMEM((_N_GRAPHS,), jnp.float32),
            pltpu.SemaphoreType.DMA,
            pltpu.SemaphoreType.DMA,
        ],
    )


def _tc_finish(part_ref, n_ref, gs_ref, gsh_ref, out_ref):
    p = part_ref[...]
    sc_sum = jnp.sum(p[0:_NW, :], axis=0, keepdims=True)
    sh_sum = jnp.sum(p[_NW:2 * _NW, :], axis=0, keepdims=True)
    e_sum = jnp.sum(p[2 * _NW:3 * _NW, :], axis=0, keepdims=True)
    num_atoms = jnp.maximum(n_ref[...].astype(jnp.float32), 1.0)
    c = math.log(math.e - 1.0)
    sc = jax.nn.softplus(sc_sum + c) / num_atoms * jax.nn.softplus(gs_ref[0] + c)
    sh = sh_sum / num_atoms + gsh_ref[0]
    out_ref[...] = (e_sum / num_atoms) * sc + sh


def kernel(energies, scale, shift, global_scale, global_shift, species, graph_i, n_node):
    partials = _build_sc_partials()(
        energies, species.astype(jnp.int32), graph_i.astype(jnp.int32), scale, shift
    )

    out = pl.pallas_call(
        _tc_finish,
        out_shape=jax.ShapeDtypeStruct((1, _N_GRAPHS), jnp.float32),
        in_specs=[
            pl.BlockSpec(memory_space=pltpu.VMEM),
            pl.BlockSpec(memory_space=pltpu.VMEM),
            pl.BlockSpec(memory_space=pltpu.SMEM),
            pl.BlockSpec(memory_space=pltpu.SMEM),
        ],
        out_specs=pl.BlockSpec(memory_space=pltpu.VMEM),
    )(partials, n_node.reshape(1, _N_GRAPHS), global_scale, global_shift)

    return out.reshape(_N_GRAPHS, 1)
